# Initial kernel scaffold; baseline (speedup 1.0000x reference)
import jax
import jax.numpy as jnp
from jax.experimental import pallas as pl


def kernel(x, mp1_edge_index, mp2_edge_index, W_proj, b_proj, W_mp1, b_mp1, W_mp2, b_mp2, W_att, b_att, a_att):
    raise NotImplementedError("write your pallas kernel here")


# speedup vs baseline: 1.9572x; 1.9572x over previous
import functools

import numpy as np

import jax
import jax.numpy as jnp
from jax import lax
from jax.experimental import pallas as pl
from jax.experimental.pallas import tpu as pltpu

_Z = np.int32(0)
N = 10000
D_IN = 256
H = 512
E = 160000

RB = 400
GRID_N = N // RB


def _proj_body(x_ref, wp_ref, bp_ref, z_ref):
    acc = jnp.dot(x_ref[...], wp_ref[...], preferred_element_type=jnp.float32)
    acc = acc + bp_ref[...]
    z_ref[...] = jnp.where(acc > 0, acc, jnp.exp(acc) - 1.0)


def _proj(x, W_proj, b_proj):
    return pl.pallas_call(
        _proj_body,
        grid=(GRID_N,),
        in_specs=[
            pl.BlockSpec((RB, D_IN), lambda i: (i, _Z)),
            pl.BlockSpec((D_IN, H), lambda i: (_Z, _Z)),
            pl.BlockSpec((H,), lambda i: (_Z,)),
        ],
        out_specs=pl.BlockSpec((RB, H), lambda i: (i, _Z)),
        out_shape=jax.ShapeDtypeStruct((N, H), jnp.float32),
    )(x, W_proj, b_proj)


def _scale_body(deg_ref, z_ref, zs_ref, nd_ref):
    deg = deg_ref[...]
    inv = lax.rsqrt(jnp.maximum(deg, 1.0))
    z = z_ref[...]
    zs_ref[0] = z * inv[:, 0][:, None]
    zs_ref[1] = z * inv[:, 2][:, None]
    nd_ref[...] = jnp.concatenate([inv[:, 1:2], inv[:, 3:4]], axis=1)


def _scale(deg, z):
    RBS = 2000
    zs, nd = pl.pallas_call(
        _scale_body,
        grid=(N // RBS,),
        in_specs=[
            pl.BlockSpec((RBS, 4), lambda i: (i, _Z)),
            pl.BlockSpec((RBS, H), lambda i: (i, _Z)),
        ],
        out_specs=[
            pl.BlockSpec((2, RBS, H), lambda i: (_Z, i, _Z)),
            pl.BlockSpec((RBS, 2), lambda i: (i, _Z)),
        ],
        out_shape=[
            jax.ShapeDtypeStruct((2, N, H), jnp.float32),
            jax.ShapeDtypeStruct((N, 2), jnp.float32),
        ],
    )(deg, z)
    return zs, nd


def _mp_body(agg1_ref, agg2_ref, nd_ref, w1_ref, b1_ref, w2_ref, b2_ref,
             wa_ref, ba_ref, aa_ref, h1_ref, h2_ref, wpart_ref):
    nd = nd_ref[...]
    parts = []
    for m, (agg_ref, w_ref, b_ref, h_ref) in enumerate(
            ((agg1_ref, w1_ref, b1_ref, h1_ref),
             (agg2_ref, w2_ref, b2_ref, h2_ref))):
        s = agg_ref[...] * nd[:, m][:, None]
        acc = jnp.dot(s, w_ref[...], preferred_element_type=jnp.float32)
        acc = acc + b_ref[...]
        h = jnp.where(acc > 0, acc, jnp.exp(acc) - 1.0)
        h_ref[...] = h
        t = jnp.dot(h, wa_ref[...], preferred_element_type=jnp.float32)
        t = jnp.tanh(t + ba_ref[...])
        parts.append(jnp.sum(t * aa_ref[...]))
    wpart_ref[...] = jnp.stack(parts).reshape(1, 1, 2)


def _mp(agg1, agg2, nd, W_mp1, b_mp1, W_mp2, b_mp2, W_att, b_att, a_att):
    full = lambda i: (_Z, _Z)
    h1, h2, wparts = pl.pallas_call(
        _mp_body,
        grid=(GRID_N,),
        in_specs=[
            pl.BlockSpec((RB, H), lambda i: (i, _Z)),
            pl.BlockSpec((RB, H), lambda i: (i, _Z)),
            pl.BlockSpec((RB, 2), lambda i: (i, _Z)),
            pl.BlockSpec((H, H), full),
            pl.BlockSpec((H,), lambda i: (_Z,)),
            pl.BlockSpec((H, H), full),
            pl.BlockSpec((H,), lambda i: (_Z,)),
            pl.BlockSpec((H, H), full),
            pl.BlockSpec((H,), lambda i: (_Z,)),
            pl.BlockSpec((H,), lambda i: (_Z,)),
        ],
        out_specs=[
            pl.BlockSpec((RB, H), lambda i: (i, _Z)),
            pl.BlockSpec((RB, H), lambda i: (i, _Z)),
            pl.BlockSpec((1, 1, 2), lambda i: (i, _Z, _Z)),
        ],
        out_shape=[
            jax.ShapeDtypeStruct((N, H), jnp.float32),
            jax.ShapeDtypeStruct((N, H), jnp.float32),
            jax.ShapeDtypeStruct((GRID_N, 1, 2), jnp.float32),
        ],
    )(agg1, agg2, nd, W_mp1, b_mp1, W_mp2, b_mp2, W_att, b_att, a_att)
    return h1, h2, wparts


def _comb_body(wparts_ref, h1_ref, h2_ref, out_ref):
    w = jnp.sum(wparts_ref[...], axis=(0, 1)) / N
    w = w - jnp.max(w)
    e = jnp.exp(w)
    beta = e / jnp.sum(e)
    out_ref[...] = beta[0] * h1_ref[...] + beta[1] * h2_ref[...]


def _combine(wparts, h1, h2):
    return pl.pallas_call(
        _comb_body,
        grid=(GRID_N,),
        in_specs=[
            pl.BlockSpec((GRID_N, 1, 2), lambda i: (_Z, _Z, _Z)),
            pl.BlockSpec((RB, H), lambda i: (i, _Z)),
            pl.BlockSpec((RB, H), lambda i: (i, _Z)),
        ],
        out_specs=pl.BlockSpec((RB, H), lambda i: (i, _Z)),
        out_shape=jax.ShapeDtypeStruct((N, H), jnp.float32),
    )(wparts, h1, h2)


def kernel(x, mp1_edge_index, mp2_edge_index, W_proj, b_proj, W_mp1, b_mp1,
           W_mp2, b_mp2, W_att, b_att, a_att):
    e1 = mp1_edge_index.astype(jnp.int32)
    e2 = mp2_edge_index.astype(jnp.int32)

    z = _proj(x, W_proj, b_proj)

    ones = jnp.ones((E,), jnp.float32)
    zeros_n = jnp.zeros((N,), jnp.float32)
    deg = jnp.stack([
        zeros_n.at[e1[0]].add(ones),
        zeros_n.at[e1[1]].add(ones),
        zeros_n.at[e2[0]].add(ones),
        zeros_n.at[e2[1]].add(ones),
    ], axis=1)
    zs, nd = _scale(deg, z)
    agg1 = jnp.zeros((N, H), jnp.float32).at[e1[1]].add(zs[0][e1[0]])
    agg2 = jnp.zeros((N, H), jnp.float32).at[e2[1]].add(zs[1][e2[0]])

    h1, h2, wparts = _mp(agg1, agg2, nd, W_mp1, b_mp1, W_mp2, b_mp2,
                         W_att, b_att, a_att)
    return _combine(wparts, h1, h2)


# speedup vs baseline: 2.0054x; 1.0246x over previous
import functools

import numpy as np

import jax
import jax.numpy as jnp
from jax import lax
from jax.experimental import pallas as pl
from jax.experimental.pallas import tpu as pltpu
from jax.experimental.pallas import tpu_sc as plsc

_Z = np.int32(0)
N = 10000
D_IN = 256
H = 512
E = 160000

RB = 400
GRID_N = N // RB


NC = 2
NS = 16
NW = NC * NS
EPT = E // NW
_DEG_FULL = EPT // 16
_DEG_REM = EPT - _DEG_FULL * 16

_SC_MESH = plsc.VectorSubcoreMesh(core_axis_name="c", subcore_axis_name="s")


def _deg_body(edges_hbm, out_hbm, ev, hist):
    cid = lax.axis_index("c")
    sid = lax.axis_index("s")
    wid = sid * NC + cid
    base = wid * EPT
    for a in range(4):
        pltpu.sync_copy(edges_hbm.at[pl.ds(a * E + base, EPT)],
                        ev.at[pl.ds(a * EPT, EPT)])
    zeros16 = jnp.zeros((16,), jnp.float32)

    def zset(i, carry):
        hist[pl.ds(i * 16, 16)] = zeros16
        return carry

    lax.fori_loop(jnp.int32(0), jnp.int32(4 * N // 16), zset, jnp.int32(0),
                  unroll=False)
    ones16 = jnp.ones((16,), jnp.float32)
    lane = lax.iota(jnp.int32, 16)
    four = jnp.int32(4)
    for a in range(4):
        off = jnp.int32(a)

        def body(k, carry):
            idx = ev[pl.ds(a * EPT + k * 16, 16)]
            plsc.addupdate_scatter(hist, [idx * four + off], ones16)
            return carry

        lax.fori_loop(jnp.int32(0), jnp.int32(_DEG_FULL), body, jnp.int32(0),
                      unroll=False)
        if _DEG_REM:
            idx = ev[pl.ds(a * EPT + _DEG_FULL * 16, 16)]
            plsc.addupdate_scatter(hist, [idx * four + off], ones16,
                                   mask=lane < _DEG_REM)
    pltpu.sync_copy(hist, out_hbm.at[wid])


def _degrees(edges_flat):
    return pl.kernel(
        _deg_body,
        out_type=jax.ShapeDtypeStruct((NW, 4 * N), jnp.float32),
        mesh=_SC_MESH,
        scratch_types=[
            pltpu.VMEM((4 * EPT + 16,), jnp.int32),
            pltpu.VMEM((4 * N,), jnp.float32),
        ],
        compiler_params=pltpu.CompilerParams(needs_layout_passes=False),
    )(edges_flat)


OWN = 160
PASSES = 2
NPAD = PASSES * NW * OWN
ESEC = 2000
_SEC_VSTEPS = ESEC // 16
NSEC = E // ESEC
BG = 16
FCAP = ESEC + BG


def _agg_body(src_hbm, dst_hbm, zs_hbm, out_hbm,
              ev_s0, ev_s1, ev_d0, ev_d1, fs, fd, gbuf, acc,
              gsem0, gsem1, esem0, esem1):
    cid = lax.axis_index("c")
    sid = lax.axis_index("s")
    wid = cid * NS + sid

    zeros16 = jnp.zeros((16,), jnp.float32)

    def eissue(sec, slot, sem):
        evs = ev_s0 if slot == 0 else ev_s1
        evd = ev_d0 if slot == 0 else ev_d1
        pltpu.async_copy(src_hbm.at[pl.ds(sec * ESEC, ESEC)], evs, sem)
        pltpu.async_copy(dst_hbm.at[pl.ds(sec * ESEC, ESEC)], evd, sem)

    def ewait(slot, sem):
        evs = ev_s0 if slot == 0 else ev_s1
        evd = ev_d0 if slot == 0 else ev_d1
        pltpu.make_async_copy(src_hbm.at[pl.ds(0, ESEC)], evs, sem).wait()
        pltpu.make_async_copy(dst_hbm.at[pl.ds(0, ESEC)], evd, sem).wait()

    def gissue(j, slot, sem):
        pltpu.async_copy(zs_hbm.at[fs.at[pl.ds(j * BG, BG)]],
                         gbuf.at[pl.ds(slot * BG, BG)], sem)

    def gwait(slot, sem):
        pltpu.make_async_copy(zs_hbm.at[pl.ds(0, BG)],
                              gbuf.at[pl.ds(slot * BG, BG)], sem).wait()

    def accumulate(j, slot, sem):
        dvec = fd[pl.ds(j * BG, 16)]
        gwait(slot, sem)
        for e in range(BG):
            dl = dvec[e]
            grow = e + slot * BG
            for c in range(H // 16):
                x = gbuf[grow, pl.ds(c * 16, 16)]
                plsc.addupdate(acc.at[dl, pl.ds(c * 16, 16)], x)

    def one_pass(p, pcarry):
        base = (p * NW + wid) * OWN

        def zrow(i, carry):
            for c in range(H // 16):
                acc[i, pl.ds(c * 16, 16)] = zeros16
            return carry

        lax.fori_loop(jnp.int32(0), jnp.int32(OWN + 4), zrow, jnp.int32(0),
                      unroll=False)

        trash16 = jnp.full((16,), OWN, jnp.int32)
        zsrc16 = jnp.zeros((16,), jnp.int32)

        def make_fbody(evs, evd):
            def fbody(k, pos):
                d = evd[pl.ds(k * 16, 16)]
                sv = evs[pl.ds(k * 16, 16)]
                dl = d - base
                m = (dl >= 0) & (dl < OWN)
                plsc.store_compressed(fd.at[pl.ds(pos, 16)], dl, mask=m)
                plsc.store_compressed(fs.at[pl.ds(pos, 16)], sv, mask=m)
                cnt = plsc.all_reduce_population_count(m)
                return pos + cnt[0]
            return fbody

        def section(sec, slot):
            esem = esem0 if slot == 0 else esem1

            @pl.when(sec + 1 < NSEC)
            def _():
                eissue(sec + 1, 1 - slot, esem1 if slot == 0 else esem0)

            ewait(slot, esem)
            pos = lax.fori_loop(
                jnp.int32(0), jnp.int32(_SEC_VSTEPS),
                make_fbody(ev_s0 if slot == 0 else ev_s1,
                           ev_d0 if slot == 0 else ev_d1),
                jnp.int32(0))
            for t in range(BG // 16):
                fd[pl.ds(pos + t * 16, 16)] = trash16
                fs[pl.ds(pos + t * 16, 16)] = zsrc16
            nb = (pos + (BG - 1)) // BG

            @pl.when(nb > 0)
            def _():
                gissue(0, 0, gsem0)

            def gpair(q, carry):
                j0 = q * 2
                j1 = j0 + 1

                @pl.when(j1 < nb)
                def _():
                    gissue(j1, 1, gsem1)

                accumulate(j0, 0, gsem0)

                @pl.when(j1 < nb)
                def _():
                    @pl.when(j1 + 1 < nb)
                    def _():
                        gissue(j1 + 1, 0, gsem0)

                    accumulate(j1, 1, gsem1)

                return carry

            lax.fori_loop(jnp.int32(0), (nb + 1) // 2, gpair, jnp.int32(0),
                          unroll=False)

        eissue(0, 0, esem0)

        def spair(pp, carry):
            section(pp * 2, 0)
            section(pp * 2 + 1, 1)
            return carry

        lax.fori_loop(jnp.int32(0), jnp.int32(NSEC // 2), spair,
                      jnp.int32(0), unroll=False)

        pltpu.sync_copy(acc.at[pl.ds(0, OWN)], out_hbm.at[pl.ds(base, OWN)])
        return pcarry

    lax.fori_loop(jnp.int32(0), jnp.int32(PASSES), one_pass, jnp.int32(0),
                  unroll=False)


def _aggregate(src, dst, zs_m):
    return pl.kernel(
        _agg_body,
        out_type=jax.ShapeDtypeStruct((NPAD, H), jnp.float32),
        mesh=_SC_MESH,
        scratch_types=[
            pltpu.VMEM((ESEC,), jnp.int32),
            pltpu.VMEM((ESEC,), jnp.int32),
            pltpu.VMEM((ESEC,), jnp.int32),
            pltpu.VMEM((ESEC,), jnp.int32),
            pltpu.VMEM((FCAP,), jnp.int32),
            pltpu.VMEM((FCAP,), jnp.int32),
            pltpu.VMEM((2 * BG, H), jnp.float32),
            pltpu.VMEM((OWN + 4, H), jnp.float32),
            pltpu.SemaphoreType.DMA,
            pltpu.SemaphoreType.DMA,
            pltpu.SemaphoreType.DMA,
            pltpu.SemaphoreType.DMA,
        ],
        compiler_params=pltpu.CompilerParams(needs_layout_passes=False),
    )(src, dst, zs_m)


def _proj_body(x_ref, wp_ref, bp_ref, z_ref):
    acc = jnp.dot(x_ref[...], wp_ref[...], preferred_element_type=jnp.float32)
    acc = acc + bp_ref[...]
    z_ref[...] = jnp.where(acc > 0, acc, jnp.exp(acc) - 1.0)


def _proj(x, W_proj, b_proj):
    return pl.pallas_call(
        _proj_body,
        grid=(GRID_N,),
        in_specs=[
            pl.BlockSpec((RB, D_IN), lambda i: (i, _Z)),
            pl.BlockSpec((D_IN, H), lambda i: (_Z, _Z)),
            pl.BlockSpec((H,), lambda i: (_Z,)),
        ],
        out_specs=pl.BlockSpec((RB, H), lambda i: (i, _Z)),
        out_shape=jax.ShapeDtypeStruct((N, H), jnp.float32),
    )(x, W_proj, b_proj)


RBS = 2000


def _norms_body(degp_ref, inv_ref):
    deg = jnp.sum(degp_ref[...], axis=0)
    inv_ref[...] = lax.rsqrt(jnp.maximum(deg, 1.0))


def _norms(deg_parts):
    return pl.pallas_call(
        _norms_body,
        grid=(1,),
        in_specs=[pl.BlockSpec((NW, 4 * N), lambda i: (_Z, _Z))],
        out_specs=pl.BlockSpec((4 * N,), lambda i: (_Z,)),
        out_shape=jax.ShapeDtypeStruct((4 * N,), jnp.float32),
    )(deg_parts)


def _scale_body(inv_ref, z_ref, zs_ref, nd_ref):
    inv = inv_ref[...]
    z = z_ref[...]
    zs_ref[0] = z * inv[:, 0][:, None]
    zs_ref[1] = z * inv[:, 2][:, None]
    nd_ref[...] = jnp.concatenate([inv[:, 1:2], inv[:, 3:4]], axis=1)


def _scale(inv_n4, z):
    zs, nd = pl.pallas_call(
        _scale_body,
        grid=(N // RBS,),
        in_specs=[
            pl.BlockSpec((RBS, 4), lambda i: (i, _Z)),
            pl.BlockSpec((RBS, H), lambda i: (i, _Z)),
        ],
        out_specs=[
            pl.BlockSpec((2, RBS, H), lambda i: (_Z, i, _Z)),
            pl.BlockSpec((RBS, 2), lambda i: (i, _Z)),
        ],
        out_shape=[
            jax.ShapeDtypeStruct((2, N, H), jnp.float32),
            jax.ShapeDtypeStruct((N, 2), jnp.float32),
        ],
    )(inv_n4, z)
    return zs, nd


def _mp_body(agg1_ref, agg2_ref, nd_ref, w1_ref, b1_ref, w2_ref, b2_ref,
             wa_ref, ba_ref, aa_ref, h1_ref, h2_ref, wpart_ref):
    nd = nd_ref[...]
    parts = []
    for m, (agg_ref, w_ref, b_ref, h_ref) in enumerate(
            ((agg1_ref, w1_ref, b1_ref, h1_ref),
             (agg2_ref, w2_ref, b2_ref, h2_ref))):
        s = agg_ref[...] * nd[:, m][:, None]
        acc = jnp.dot(s, w_ref[...], preferred_element_type=jnp.float32)
        acc = acc + b_ref[...]
        h = jnp.where(acc > 0, acc, jnp.exp(acc) - 1.0)
        h_ref[...] = h
        t = jnp.dot(h, wa_ref[...], preferred_element_type=jnp.float32)
        t = jnp.tanh(t + ba_ref[...])
        parts.append(jnp.sum(t * aa_ref[...]))
    wpart_ref[...] = jnp.stack(parts).reshape(1, 1, 2)


def _mp(agg1, agg2, nd, W_mp1, b_mp1, W_mp2, b_mp2, W_att, b_att, a_att):
    full = lambda i: (_Z, _Z)
    h1, h2, wparts = pl.pallas_call(
        _mp_body,
        grid=(GRID_N,),
        in_specs=[
            pl.BlockSpec((RB, H), lambda i: (i, _Z)),
            pl.BlockSpec((RB, H), lambda i: (i, _Z)),
            pl.BlockSpec((RB, 2), lambda i: (i, _Z)),
            pl.BlockSpec((H, H), full),
            pl.BlockSpec((H,), lambda i: (_Z,)),
            pl.BlockSpec((H, H), full),
            pl.BlockSpec((H,), lambda i: (_Z,)),
            pl.BlockSpec((H, H), full),
            pl.BlockSpec((H,), lambda i: (_Z,)),
            pl.BlockSpec((H,), lambda i: (_Z,)),
        ],
        out_specs=[
            pl.BlockSpec((RB, H), lambda i: (i, _Z)),
            pl.BlockSpec((RB, H), lambda i: (i, _Z)),
            pl.BlockSpec((1, 1, 2), lambda i: (i, _Z, _Z)),
        ],
        out_shape=[
            jax.ShapeDtypeStruct((N, H), jnp.float32),
            jax.ShapeDtypeStruct((N, H), jnp.float32),
            jax.ShapeDtypeStruct((GRID_N, 1, 2), jnp.float32),
        ],
    )(agg1, agg2, nd, W_mp1, b_mp1, W_mp2, b_mp2, W_att, b_att, a_att)
    return h1, h2, wparts


def _comb_body(wparts_ref, h1_ref, h2_ref, out_ref):
    w = jnp.sum(wparts_ref[...], axis=(0, 1)) / N
    w = w - jnp.max(w)
    e = jnp.exp(w)
    beta = e / jnp.sum(e)
    out_ref[...] = beta[0] * h1_ref[...] + beta[1] * h2_ref[...]


def _combine(wparts, h1, h2):
    return pl.pallas_call(
        _comb_body,
        grid=(GRID_N,),
        in_specs=[
            pl.BlockSpec((GRID_N, 1, 2), lambda i: (_Z, _Z, _Z)),
            pl.BlockSpec((RB, H), lambda i: (i, _Z)),
            pl.BlockSpec((RB, H), lambda i: (i, _Z)),
        ],
        out_specs=pl.BlockSpec((RB, H), lambda i: (i, _Z)),
        out_shape=jax.ShapeDtypeStruct((N, H), jnp.float32),
    )(wparts, h1, h2)


def kernel(x, mp1_edge_index, mp2_edge_index, W_proj, b_proj, W_mp1, b_mp1,
           W_mp2, b_mp2, W_att, b_att, a_att):
    e1 = mp1_edge_index.astype(jnp.int32)
    e2 = mp2_edge_index.astype(jnp.int32)

    z = _proj(x, W_proj, b_proj)

    edges_flat = jnp.concatenate([e1[0], e1[1], e2[0], e2[1]])
    deg_parts = _degrees(edges_flat)
    inv_n4 = _norms(deg_parts).reshape(N, 4)
    zs, nd = _scale(inv_n4, z)

    agg1 = _aggregate(e1[0], e1[1], zs[0])[:N]
    agg2 = _aggregate(e2[0], e2[1], zs[1])[:N]

    h1, h2, wparts = _mp(agg1, agg2, nd, W_mp1, b_mp1, W_mp2, b_mp2,
                         W_att, b_att, a_att)
    return _combine(wparts, h1, h2)
